# baseline (speedup 1.0000x reference)
import functools

import numpy as np
import jax
import jax.numpy as jnp
from jax import lax
from jax.experimental import pallas as pl
from jax.experimental.pallas import tpu as pltpu
from jax.experimental.pallas import tpu_sc as plsc

_KS2 = np.int32(0x1BD11BDA)
_MANT_ONE = np.int32(0x3F800000)
_TINY = np.float32(np.finfo(np.float32).tiny)
_BIG = np.int32(np.iinfo(np.int32).max)

_WIDTH = 16384
_SUB = 256
_SC_WORKERS = 32
_SC_CHUNK = 16384


def _rotl(v, r):
    return (v << np.int32(r)) | lax.shift_right_logical(v, np.int32(32 - r))


def _threefry_bits(j):
    x0 = j
    x1 = _rotl(j, 13) ^ j
    for r in (15, 26, 6):
        x0 = x0 + x1
        x1 = _rotl(x1, r)
        x1 = x1 ^ x0
    x1 = x1 + np.int32(_KS2 + 1)
    for r in (17, 29, 16, 24):
        x0 = x0 + x1
        x1 = _rotl(x1, r)
        x1 = x1 ^ x0
    x0 = x0 + _KS2
    x1 = x1 + np.int32(2)
    for r in (13, 15, 26, 6):
        x0 = x0 + x1
        x1 = _rotl(x1, r)
        x1 = x1 ^ x0
    x1 = x1 + np.int32(3)
    for r in (17, 29, 16, 24):
        x0 = x0 + x1
        x1 = _rotl(x1, r)
        x1 = x1 ^ x0
    x1 = x1 + np.int32(_KS2 + 4)
    for r in (13, 15, 26, 6):
        x0 = x0 + x1
        x1 = _rotl(x1, r)
        x1 = x1 ^ x0
    x0 = x0 + _KS2
    x1 = x1 + np.int32(5)
    return x0 ^ x1


def _gumbel_from_bits(bits):
    mant = lax.shift_right_logical(bits, np.int32(9)) | _MANT_ONE
    u = lax.bitcast_convert_type(mant, jnp.float32) - np.float32(1.0)
    u = u * (np.float32(1.0) - _TINY) + _TINY
    u = jnp.maximum(_TINY, u)
    return -jnp.log(-jnp.log(u))


def _fold_step(k, loc_max, loc_idx, max_ref, idx_ref):
    @pl.when(k == 0)
    def _():
        max_ref[...] = loc_max
        idx_ref[...] = loc_idx

    @pl.when(k > 0)
    def _():
        upd = loc_max > max_ref[...]
        max_ref[...] = jnp.where(upd, loc_max, max_ref[...])
        idx_ref[...] = jnp.where(upd, loc_idx, idx_ref[...])


def _reduce_block(ymax, argj, rows, ncols):
    loc_max = jnp.max(ymax, axis=1, keepdims=True)
    loc_j = jnp.min(jnp.where(ymax == loc_max, argj, _BIG),
                    axis=1, keepdims=True)
    loc_idx = loc_j - (lax.broadcasted_iota(jnp.int32, (rows, 1), 0)
                       * np.int32(ncols))
    return loc_max, loc_idx


def _body_gen(x_ref, pmax_ref, pidx_ref, max_ref, idx_ref,
              *, ncols, width, sub, col_base):
    k = pl.program_id(0)
    nsteps = pl.num_programs(0)
    rows = x_ref.shape[0]
    c0 = np.int32(col_base) + k * np.int32(width)

    lane = lax.broadcasted_iota(jnp.int32, (rows, sub), 1)
    row_base = lax.broadcasted_iota(jnp.int32, (rows, sub), 0) * np.int32(ncols)
    j0 = row_base + (lane + c0)
    row_limit = row_base + np.int32(ncols)

    def inner(i, carry):
        ymax, argj = carry
        start = pl.multiple_of(i * sub, sub)
        xs = x_ref[:, pl.ds(start, sub)]
        jv = j0 + i * np.int32(sub)
        y = _gumbel_from_bits(_threefry_bits(jv)) + xs
        upd = (y > ymax) & (jv < row_limit)
        return jnp.where(upd, y, ymax), jnp.where(upd, jv, argj)

    init = (jnp.full((rows, sub), -jnp.inf, jnp.float32),
            jnp.zeros((rows, sub), jnp.int32))
    ymax, argj = lax.fori_loop(0, width // sub, inner, init)

    loc_max, loc_idx = _reduce_block(ymax, argj, rows, ncols)
    _fold_step(k, loc_max, loc_idx, max_ref, idx_ref)

    @pl.when(k == nsteps - 1)
    def _():
        pmax_ref[...] = max_ref[...]
        pidx_ref[...] = idx_ref[...]


def _body_bits(x_ref, bits_ref, bmax_ref, bidx_ref, out_ref,
               max_ref, idx_ref, *, ncols, width, sub):
    k = pl.program_id(0)
    nsteps = pl.num_programs(0)
    rows = x_ref.shape[0]
    c0 = k * np.int32(width)

    lane = lax.broadcasted_iota(jnp.int32, (rows, sub), 1)
    row_base = lax.broadcasted_iota(jnp.int32, (rows, sub), 0) * np.int32(ncols)
    j0 = row_base + (lane + c0)

    def inner(i, carry):
        ymax, argj = carry
        start = pl.multiple_of(i * sub, sub)
        xs = x_ref[:, pl.ds(start, sub)]
        bits = bits_ref[:, pl.ds(start, sub)]
        y = _gumbel_from_bits(bits) + xs
        jv = j0 + i * np.int32(sub)
        upd = y > ymax
        return jnp.where(upd, y, ymax), jnp.where(upd, jv, argj)

    init = (jnp.full((rows, sub), -jnp.inf, jnp.float32),
            jnp.zeros((rows, sub), jnp.int32))
    ymax, argj = lax.fori_loop(0, width // sub, inner, init)

    loc_max, loc_idx = _reduce_block(ymax, argj, rows, ncols)
    _fold_step(k, loc_max, loc_idx, max_ref, idx_ref)

    @pl.when(k == nsteps - 1)
    def _():
        take_b = bmax_ref[...] > max_ref[...]
        out_ref[...] = jnp.where(take_b, bidx_ref[...], idx_ref[...])


def _sc_bits(rows, ncols, ca):
    mesh = plsc.VectorSubcoreMesh(core_axis_name="c", subcore_axis_name="s")
    rows_per_w = rows // _SC_WORKERS
    nchunks = ca // _SC_CHUNK

    @functools.partial(
        pl.kernel,
        mesh=mesh,
        out_type=jax.ShapeDtypeStruct((rows, ca), jnp.int32),
        scratch_types=[pltpu.VMEM((_SC_CHUNK,), jnp.int32)],
    )
    def k(out_hbm, stage):
        wid = lax.axis_index("s") * np.int32(2) + lax.axis_index("c")
        iota = lax.broadcasted_iota(jnp.int32, (16,), 0)

        def row_body(rl, _):
            row = wid * np.int32(rows_per_w) + rl

            def chunk_body(cb, _):
                jbase = row * np.int32(ncols) + cb * np.int32(_SC_CHUNK)

                def vec_body(ci, _):
                    base = ci * np.int32(128)
                    for off in range(8):
                        o = base + np.int32(off * 16)
                        stage[pl.ds(o, 16)] = _threefry_bits(iota + (jbase + o))
                    return 0

                lax.fori_loop(0, _SC_CHUNK // 128, vec_body, 0)
                pltpu.sync_copy(
                    stage, out_hbm.at[row, pl.ds(cb * np.int32(_SC_CHUNK),
                                                 _SC_CHUNK)])
                return 0

            return lax.fori_loop(0, nchunks, chunk_body, 0)

        lax.fori_loop(0, rows_per_w, row_body, 0)

    return k


def _tc_partial(x, ncols, col_base, span):
    rows = x.shape[0]
    nsteps = pl.cdiv(span, _WIDTH)
    kb = col_base // _WIDTH
    return pl.pallas_call(
        functools.partial(_body_gen, ncols=ncols, width=_WIDTH, sub=_SUB,
                          col_base=col_base),
        grid=(nsteps,),
        in_specs=[pl.BlockSpec((rows, _WIDTH), lambda k: (0, k + kb))],
        out_specs=[pl.BlockSpec((rows, 1), lambda k: (0, 0)),
                   pl.BlockSpec((rows, 1), lambda k: (0, 0))],
        out_shape=[jax.ShapeDtypeStruct((rows, 1), jnp.float32),
                   jax.ShapeDtypeStruct((rows, 1), jnp.int32)],
        scratch_shapes=[
            pltpu.VMEM((rows, 1), jnp.float32),
            pltpu.VMEM((rows, 1), jnp.int32),
        ],
    )(x)


def kernel(x):
    rows, ncols = x.shape
    ca = 311296
    hybrid = (rows % _SC_WORKERS == 0 and ncols >= 2 * ca
              and ca % _WIDTH == 0 and ca % _SC_CHUNK == 0)

    if not hybrid:
        _, pidx = _tc_partial(x, ncols, 0, ncols)
        return pidx.reshape(rows)

    bmax, bidx = _tc_partial(x, ncols, ca, ncols - ca)
    bits = _sc_bits(rows, ncols, ca)()
    out = pl.pallas_call(
        functools.partial(_body_bits, ncols=ncols, width=_WIDTH, sub=_SUB),
        grid=(ca // _WIDTH,),
        in_specs=[pl.BlockSpec((rows, _WIDTH), lambda k: (0, k)),
                  pl.BlockSpec((rows, _WIDTH), lambda k: (0, k)),
                  pl.BlockSpec((rows, 1), lambda k: (0, 0)),
                  pl.BlockSpec((rows, 1), lambda k: (0, 0))],
        out_specs=pl.BlockSpec((rows, 1), lambda k: (0, 0)),
        out_shape=jax.ShapeDtypeStruct((rows, 1), jnp.int32),
        scratch_shapes=[
            pltpu.VMEM((rows, 1), jnp.float32),
            pltpu.VMEM((rows, 1), jnp.int32),
        ],
    )(x, bits, bmax, bidx)
    return out.reshape(rows)


# speedup vs baseline: 1.0352x; 1.0352x over previous
import functools

import numpy as np
import jax
import jax.numpy as jnp
from jax import lax
from jax.experimental import pallas as pl
from jax.experimental.pallas import tpu as pltpu
from jax.experimental.pallas import tpu_sc as plsc

_KS2 = np.int32(0x1BD11BDA)
_MANT_ONE = np.int32(0x3F800000)
_TINY = np.float32(np.finfo(np.float32).tiny)
_BIG = np.int32(np.iinfo(np.int32).max)

_WIDTH = 16384
_SUB = 256
_SC_WORKERS = 32
_SC_CHUNK = 16384


def _rotl(v, r):
    return (v << np.int32(r)) | lax.shift_right_logical(v, np.int32(32 - r))


def _threefry_bits(j):
    x0 = j
    x1 = _rotl(j, 13) ^ j
    for r in (15, 26, 6):
        x0 = x0 + x1
        x1 = _rotl(x1, r)
        x1 = x1 ^ x0
    x1 = x1 + np.int32(_KS2 + 1)
    for r in (17, 29, 16, 24):
        x0 = x0 + x1
        x1 = _rotl(x1, r)
        x1 = x1 ^ x0
    x0 = x0 + _KS2
    x1 = x1 + np.int32(2)
    for r in (13, 15, 26, 6):
        x0 = x0 + x1
        x1 = _rotl(x1, r)
        x1 = x1 ^ x0
    x1 = x1 + np.int32(3)
    for r in (17, 29, 16, 24):
        x0 = x0 + x1
        x1 = _rotl(x1, r)
        x1 = x1 ^ x0
    x1 = x1 + np.int32(_KS2 + 4)
    for r in (13, 15, 26, 6):
        x0 = x0 + x1
        x1 = _rotl(x1, r)
        x1 = x1 ^ x0
    x0 = x0 + _KS2
    x1 = x1 + np.int32(5)
    return x0 ^ x1


def _gumbel_from_bits(bits):
    mant = lax.shift_right_logical(bits, np.int32(9)) | _MANT_ONE
    u = lax.bitcast_convert_type(mant, jnp.float32) - np.float32(1.0)
    u = u * (np.float32(1.0) - _TINY) + _TINY
    u = jnp.maximum(_TINY, u)
    return -jnp.log(-jnp.log(u))


def _fold_step(k, loc_max, loc_idx, max_ref, idx_ref):
    @pl.when(k == 0)
    def _():
        max_ref[...] = loc_max
        idx_ref[...] = loc_idx

    @pl.when(k > 0)
    def _():
        upd = loc_max > max_ref[...]
        max_ref[...] = jnp.where(upd, loc_max, max_ref[...])
        idx_ref[...] = jnp.where(upd, loc_idx, idx_ref[...])


def _reduce_block(ymax, argj, rows, ncols):
    loc_max = jnp.max(ymax, axis=1, keepdims=True)
    loc_j = jnp.min(jnp.where(ymax == loc_max, argj, _BIG),
                    axis=1, keepdims=True)
    loc_idx = loc_j - (lax.broadcasted_iota(jnp.int32, (rows, 1), 0)
                       * np.int32(ncols))
    return loc_max, loc_idx


def _body_gen(x_ref, pmax_ref, pidx_ref, max_ref, idx_ref,
              *, ncols, width, sub, col_base):
    k = pl.program_id(0)
    nsteps = pl.num_programs(0)
    rows = x_ref.shape[0]
    c0 = np.int32(col_base) + k * np.int32(width)

    lane = lax.broadcasted_iota(jnp.int32, (rows, sub), 1)
    row_base = lax.broadcasted_iota(jnp.int32, (rows, sub), 0) * np.int32(ncols)
    j0 = row_base + (lane + c0)
    row_limit = row_base + np.int32(ncols)

    def inner(i, carry):
        ymax, argj = carry
        start = pl.multiple_of(i * sub, sub)
        xs = x_ref[:, pl.ds(start, sub)]
        jv = j0 + i * np.int32(sub)
        y = _gumbel_from_bits(_threefry_bits(jv)) + xs
        upd = (y > ymax) & (jv < row_limit)
        return jnp.where(upd, y, ymax), jnp.where(upd, jv, argj)

    init = (jnp.full((rows, sub), -jnp.inf, jnp.float32),
            jnp.zeros((rows, sub), jnp.int32))
    ymax, argj = lax.fori_loop(0, width // sub, inner, init)

    loc_max, loc_idx = _reduce_block(ymax, argj, rows, ncols)
    _fold_step(k, loc_max, loc_idx, max_ref, idx_ref)

    @pl.when(k == nsteps - 1)
    def _():
        pmax_ref[...] = max_ref[...]
        pidx_ref[...] = idx_ref[...]


def _body_bits(x_ref, bits_ref, bmax_ref, bidx_ref, out_ref,
               max_ref, idx_ref, *, ncols, width, sub):
    k = pl.program_id(0)
    nsteps = pl.num_programs(0)
    rows = x_ref.shape[0]
    c0 = k * np.int32(width)

    lane = lax.broadcasted_iota(jnp.int32, (rows, sub), 1)
    row_base = lax.broadcasted_iota(jnp.int32, (rows, sub), 0) * np.int32(ncols)
    j0 = row_base + (lane + c0)

    def inner(i, carry):
        ymax, argj = carry
        start = pl.multiple_of(i * sub, sub)
        xs = x_ref[:, pl.ds(start, sub)]
        bits = bits_ref[:, pl.ds(start, sub)]
        y = _gumbel_from_bits(bits) + xs
        jv = j0 + i * np.int32(sub)
        upd = y > ymax
        return jnp.where(upd, y, ymax), jnp.where(upd, jv, argj)

    init = (jnp.full((rows, sub), -jnp.inf, jnp.float32),
            jnp.zeros((rows, sub), jnp.int32))
    ymax, argj = lax.fori_loop(0, width // sub, inner, init)

    loc_max, loc_idx = _reduce_block(ymax, argj, rows, ncols)
    _fold_step(k, loc_max, loc_idx, max_ref, idx_ref)

    @pl.when(k == nsteps - 1)
    def _():
        take_b = bmax_ref[...] > max_ref[...]
        out_ref[...] = jnp.where(take_b, bidx_ref[...], idx_ref[...])


def _sc_bits(rows, ncols, ca):
    mesh = plsc.VectorSubcoreMesh(core_axis_name="c", subcore_axis_name="s")
    rows_per_w = rows // _SC_WORKERS
    nchunks = ca // _SC_CHUNK

    @functools.partial(
        pl.kernel,
        mesh=mesh,
        out_type=jax.ShapeDtypeStruct((rows, ca), jnp.int32),
        scratch_types=[pltpu.VMEM((_SC_CHUNK,), jnp.int32)],
    )
    def k(out_hbm, stage):
        wid = lax.axis_index("s") * np.int32(2) + lax.axis_index("c")
        iota = lax.broadcasted_iota(jnp.int32, (16,), 0)

        def row_body(rl, _):
            row = wid * np.int32(rows_per_w) + rl

            def chunk_body(cb, _):
                jbase = row * np.int32(ncols) + cb * np.int32(_SC_CHUNK)

                def vec_body(ci, _):
                    base = ci * np.int32(128)
                    for off in range(8):
                        o = base + np.int32(off * 16)
                        stage[pl.ds(o, 16)] = _threefry_bits(iota + (jbase + o))
                    return 0

                lax.fori_loop(0, _SC_CHUNK // 128, vec_body, 0)
                pltpu.sync_copy(
                    stage, out_hbm.at[row, pl.ds(cb * np.int32(_SC_CHUNK),
                                                 _SC_CHUNK)])
                return 0

            return lax.fori_loop(0, nchunks, chunk_body, 0)

        lax.fori_loop(0, rows_per_w, row_body, 0)

    return k


def _tc_partial(x, ncols, col_base, span):
    rows = x.shape[0]
    nsteps = pl.cdiv(span, _WIDTH)
    kb = col_base // _WIDTH
    return pl.pallas_call(
        functools.partial(_body_gen, ncols=ncols, width=_WIDTH, sub=_SUB,
                          col_base=col_base),
        grid=(nsteps,),
        in_specs=[pl.BlockSpec((rows, _WIDTH), lambda k: (0, k + kb))],
        out_specs=[pl.BlockSpec((rows, 1), lambda k: (0, 0)),
                   pl.BlockSpec((rows, 1), lambda k: (0, 0))],
        out_shape=[jax.ShapeDtypeStruct((rows, 1), jnp.float32),
                   jax.ShapeDtypeStruct((rows, 1), jnp.int32)],
        scratch_shapes=[
            pltpu.VMEM((rows, 1), jnp.float32),
            pltpu.VMEM((rows, 1), jnp.int32),
        ],
    )(x)


def kernel(x):
    rows, ncols = x.shape
    ca = 294912
    hybrid = (rows % _SC_WORKERS == 0 and ncols >= 2 * ca
              and ca % _WIDTH == 0 and ca % _SC_CHUNK == 0)

    if not hybrid:
        _, pidx = _tc_partial(x, ncols, 0, ncols)
        return pidx.reshape(rows)

    bmax, bidx = _tc_partial(x, ncols, ca, ncols - ca)
    bits = _sc_bits(rows, ncols, ca)()
    out = pl.pallas_call(
        functools.partial(_body_bits, ncols=ncols, width=_WIDTH, sub=_SUB),
        grid=(ca // _WIDTH,),
        in_specs=[pl.BlockSpec((rows, _WIDTH), lambda k: (0, k)),
                  pl.BlockSpec((rows, _WIDTH), lambda k: (0, k)),
                  pl.BlockSpec((rows, 1), lambda k: (0, 0)),
                  pl.BlockSpec((rows, 1), lambda k: (0, 0))],
        out_specs=pl.BlockSpec((rows, 1), lambda k: (0, 0)),
        out_shape=jax.ShapeDtypeStruct((rows, 1), jnp.int32),
        scratch_shapes=[
            pltpu.VMEM((rows, 1), jnp.float32),
            pltpu.VMEM((rows, 1), jnp.int32),
        ],
    )(x, bits, bmax, bidx)
    return out.reshape(rows)


# speedup vs baseline: 1.0544x; 1.0186x over previous
import functools

import numpy as np
import jax
import jax.numpy as jnp
from jax import lax
from jax.experimental import pallas as pl
from jax.experimental.pallas import tpu as pltpu
from jax.experimental.pallas import tpu_sc as plsc

_KS2 = np.int32(0x1BD11BDA)
_MANT_ONE = np.int32(0x3F800000)
_TINY = np.float32(np.finfo(np.float32).tiny)
_BIG = np.int32(np.iinfo(np.int32).max)

_WIDTH = 16384
_SUB = 384
_SC_WORKERS = 32
_SC_CHUNK = 16384


def _rotl(v, r):
    return (v << np.int32(r)) | lax.shift_right_logical(v, np.int32(32 - r))


def _threefry_bits(j):
    x0 = j
    x1 = _rotl(j, 13) ^ j
    for r in (15, 26, 6):
        x0 = x0 + x1
        x1 = _rotl(x1, r)
        x1 = x1 ^ x0
    x1 = x1 + np.int32(_KS2 + 1)
    for r in (17, 29, 16, 24):
        x0 = x0 + x1
        x1 = _rotl(x1, r)
        x1 = x1 ^ x0
    x0 = x0 + _KS2
    x1 = x1 + np.int32(2)
    for r in (13, 15, 26, 6):
        x0 = x0 + x1
        x1 = _rotl(x1, r)
        x1 = x1 ^ x0
    x1 = x1 + np.int32(3)
    for r in (17, 29, 16, 24):
        x0 = x0 + x1
        x1 = _rotl(x1, r)
        x1 = x1 ^ x0
    x1 = x1 + np.int32(_KS2 + 4)
    for r in (13, 15, 26, 6):
        x0 = x0 + x1
        x1 = _rotl(x1, r)
        x1 = x1 ^ x0
    x0 = x0 + _KS2
    x1 = x1 + np.int32(5)
    return x0 ^ x1


def _gumbel_from_bits(bits):
    mant = lax.shift_right_logical(bits, np.int32(9)) | _MANT_ONE
    u = lax.bitcast_convert_type(mant, jnp.float32) - np.float32(1.0)
    u = u * (np.float32(1.0) - _TINY) + _TINY
    u = jnp.maximum(_TINY, u)
    return -jnp.log(-jnp.log(u))


def _fold_step(k, loc_max, loc_idx, max_ref, idx_ref):
    @pl.when(k == 0)
    def _():
        max_ref[...] = loc_max
        idx_ref[...] = loc_idx

    @pl.when(k > 0)
    def _():
        upd = loc_max > max_ref[...]
        max_ref[...] = jnp.where(upd, loc_max, max_ref[...])
        idx_ref[...] = jnp.where(upd, loc_idx, idx_ref[...])


def _reduce_block(ymax, argj, rows, ncols):
    loc_max = jnp.max(ymax, axis=1, keepdims=True)
    loc_j = jnp.min(jnp.where(ymax == loc_max, argj, _BIG),
                    axis=1, keepdims=True)
    loc_idx = loc_j - (lax.broadcasted_iota(jnp.int32, (rows, 1), 0)
                       * np.int32(ncols))
    return loc_max, loc_idx


def _body_gen(x_ref, pmax_ref, pidx_ref, max_ref, idx_ref,
              *, ncols, width, sub, col_base):
    k = pl.program_id(0)
    nsteps = pl.num_programs(0)
    rows = x_ref.shape[0]
    c0 = np.int32(col_base) + k * np.int32(width)

    lane = lax.broadcasted_iota(jnp.int32, (rows, sub), 1)
    row_base = lax.broadcasted_iota(jnp.int32, (rows, sub), 0) * np.int32(ncols)
    j0 = row_base + (lane + c0)
    row_limit = row_base + np.int32(ncols)

    def inner(i, carry):
        ymax, argj = carry
        start = pl.multiple_of(i * sub, sub)
        xs = x_ref[:, pl.ds(start, sub)]
        jv = j0 + i * np.int32(sub)
        y = _gumbel_from_bits(_threefry_bits(jv)) + xs
        upd = (y > ymax) & (jv < row_limit)
        return jnp.where(upd, y, ymax), jnp.where(upd, jv, argj)

    init = (jnp.full((rows, sub), -jnp.inf, jnp.float32),
            jnp.zeros((rows, sub), jnp.int32))
    ymax, argj = lax.fori_loop(0, width // sub, inner, init)

    loc_max, loc_idx = _reduce_block(ymax, argj, rows, ncols)
    _fold_step(k, loc_max, loc_idx, max_ref, idx_ref)

    @pl.when(k == nsteps - 1)
    def _():
        pmax_ref[...] = max_ref[...]
        pidx_ref[...] = idx_ref[...]


def _body_bits(x_ref, bits_ref, bmax_ref, bidx_ref, out_ref,
               max_ref, idx_ref, *, ncols, width, sub):
    k = pl.program_id(0)
    nsteps = pl.num_programs(0)
    rows = x_ref.shape[0]
    c0 = k * np.int32(width)

    lane = lax.broadcasted_iota(jnp.int32, (rows, sub), 1)
    row_base = lax.broadcasted_iota(jnp.int32, (rows, sub), 0) * np.int32(ncols)
    j0 = row_base + (lane + c0)

    def inner(i, carry):
        ymax, argj = carry
        start = pl.multiple_of(i * sub, sub)
        xs = x_ref[:, pl.ds(start, sub)]
        bits = bits_ref[:, pl.ds(start, sub)]
        y = _gumbel_from_bits(bits) + xs
        jv = j0 + i * np.int32(sub)
        upd = y > ymax
        return jnp.where(upd, y, ymax), jnp.where(upd, jv, argj)

    init = (jnp.full((rows, sub), -jnp.inf, jnp.float32),
            jnp.zeros((rows, sub), jnp.int32))
    ymax, argj = lax.fori_loop(0, width // sub, inner, init)

    loc_max, loc_idx = _reduce_block(ymax, argj, rows, ncols)
    _fold_step(k, loc_max, loc_idx, max_ref, idx_ref)

    @pl.when(k == nsteps - 1)
    def _():
        take_b = bmax_ref[...] > max_ref[...]
        out_ref[...] = jnp.where(take_b, bidx_ref[...], idx_ref[...])


def _sc_bits(rows, ncols, ca):
    mesh = plsc.VectorSubcoreMesh(core_axis_name="c", subcore_axis_name="s")
    rows_per_w = rows // _SC_WORKERS
    nchunks = ca // _SC_CHUNK

    @functools.partial(
        pl.kernel,
        mesh=mesh,
        out_type=jax.ShapeDtypeStruct((rows, ca), jnp.int32),
        scratch_types=[pltpu.VMEM((_SC_CHUNK,), jnp.int32)],
    )
    def k(out_hbm, stage):
        wid = lax.axis_index("s") * np.int32(2) + lax.axis_index("c")
        iota = lax.broadcasted_iota(jnp.int32, (16,), 0)

        def row_body(rl, _):
            row = wid * np.int32(rows_per_w) + rl

            def chunk_body(cb, _):
                jbase = row * np.int32(ncols) + cb * np.int32(_SC_CHUNK)

                def vec_body(ci, _):
                    base = ci * np.int32(128)
                    for off in range(8):
                        o = base + np.int32(off * 16)
                        stage[pl.ds(o, 16)] = _threefry_bits(iota + (jbase + o))
                    return 0

                lax.fori_loop(0, _SC_CHUNK // 128, vec_body, 0)
                pltpu.sync_copy(
                    stage, out_hbm.at[row, pl.ds(cb * np.int32(_SC_CHUNK),
                                                 _SC_CHUNK)])
                return 0

            return lax.fori_loop(0, nchunks, chunk_body, 0)

        lax.fori_loop(0, rows_per_w, row_body, 0)

    return k


def _tc_partial(x, ncols, col_base, span):
    rows = x.shape[0]
    nsteps = pl.cdiv(span, _WIDTH)
    kb = col_base // _WIDTH
    return pl.pallas_call(
        functools.partial(_body_gen, ncols=ncols, width=_WIDTH, sub=_SUB,
                          col_base=col_base),
        grid=(nsteps,),
        in_specs=[pl.BlockSpec((rows, _WIDTH), lambda k: (0, k + kb))],
        out_specs=[pl.BlockSpec((rows, 1), lambda k: (0, 0)),
                   pl.BlockSpec((rows, 1), lambda k: (0, 0))],
        out_shape=[jax.ShapeDtypeStruct((rows, 1), jnp.float32),
                   jax.ShapeDtypeStruct((rows, 1), jnp.int32)],
        scratch_shapes=[
            pltpu.VMEM((rows, 1), jnp.float32),
            pltpu.VMEM((rows, 1), jnp.int32),
        ],
    )(x)


def kernel(x):
    rows, ncols = x.shape
    ca = 294912
    hybrid = (rows % _SC_WORKERS == 0 and ncols >= 2 * ca
              and ca % _WIDTH == 0 and ca % _SC_CHUNK == 0)

    if not hybrid:
        _, pidx = _tc_partial(x, ncols, 0, ncols)
        return pidx.reshape(rows)

    bmax, bidx = _tc_partial(x, ncols, ca, ncols - ca)
    bits = _sc_bits(rows, ncols, ca)()
    out = pl.pallas_call(
        functools.partial(_body_bits, ncols=ncols, width=_WIDTH, sub=_SUB),
        grid=(ca // _WIDTH,),
        in_specs=[pl.BlockSpec((rows, _WIDTH), lambda k: (0, k)),
                  pl.BlockSpec((rows, _WIDTH), lambda k: (0, k)),
                  pl.BlockSpec((rows, 1), lambda k: (0, 0)),
                  pl.BlockSpec((rows, 1), lambda k: (0, 0))],
        out_specs=pl.BlockSpec((rows, 1), lambda k: (0, 0)),
        out_shape=jax.ShapeDtypeStruct((rows, 1), jnp.int32),
        scratch_shapes=[
            pltpu.VMEM((rows, 1), jnp.float32),
            pltpu.VMEM((rows, 1), jnp.int32),
        ],
    )(x, bits, bmax, bidx)
    return out.reshape(rows)
